# baseline (speedup 1.0000x reference)
import jax
import jax.numpy as jnp
from jax import lax
from jax.experimental import pallas as pl
from jax.experimental.pallas import tpu as pltpu
from jax.experimental.pallas import tpu_sc as plsc

N = 100000
E = 6400000
T = 6
M = 10
R = 3
G = 8

NW = 32
CHT = 25600
NCHT = E // CHT
MAXT = -(-NCHT // NW)
SUB = 800
NSUB = CHT // SUB
NV = SUB // 16


def _sc_body(edge_hbm, xm_hbm, out_hbm, eb0, eb1, rowsb, accmat, cntb,
             sem0, sem1, semg):
    wid = lax.axis_index("s") * 2 + lax.axis_index("c")

    def any_lane(mask):
        idx = jnp.where(mask, 0, 16 + lax.iota(jnp.int32, 16))
        plsc.store_scatter(cntb.at[pl.ds(0, 32)], [idx],
                           jnp.ones((16,), jnp.int32))
        return cntb[pl.ds(0, 16)][0] > 0

    def reset_cell():
        cntb[pl.ds(0, 16)] = jnp.full((16,), -1, jnp.int32)

    reset_cell()
    accmat[...] = jnp.zeros((16,), jnp.float32)

    trips = (NCHT - wid + NW - 1) // NW
    bufs = (eb0, eb1)
    sems = (sem0, sem1)

    def chunk_slice(k):
        coff = pl.multiple_of((wid + NW * k) * CHT, 128)
        return edge_hbm.at[:, pl.ds(coff, CHT)]

    def scan_buf(ebuf):
        def sub_body(sub, carry1):
            soff = pl.multiple_of(sub * SUB, 16)

            vm = ebuf[1, pl.ds(soff, 16)]
            for j in range(1, NV):
                vm = jnp.maximum(vm, ebuf[1, pl.ds(soff + j * 16, 16)])

            @pl.when(any_lane(vm == N - 1))
            def _():
                reset_cell()

                def hit_body(j, carry2):
                    off = pl.multiple_of(soff + j * 16, 16)
                    v = ebuf[1, pl.ds(off, 16)]
                    m = v == N - 1

                    @pl.when(any_lane(m))
                    def _():
                        reset_cell()
                        sv = ebuf[0, pl.ds(off, 16)]
                        mi = m.astype(jnp.int32)
                        lanes = lax.iota(jnp.int32, 16)
                        for l in range(16):
                            @pl.when(mi[l] == 1)
                            def _():
                                svl = sv[l]
                                base = pl.multiple_of((svl // 8) * 8, 8)
                                pltpu.sync_copy(
                                    xm_hbm.at[pl.ds(base, 8), :], rowsb)
                                ridx = jnp.full((16,), svl - base, jnp.int32)
                                cidx = jnp.minimum(lanes, T - 1)
                                row = plsc.load_gather(rowsb, [ridx, cidx])
                                accmat[...] = (
                                    accmat[...]
                                    + jnp.where(lanes < T, row, 0.0))

                    return carry2

                lax.fori_loop(0, NV, hit_body, 0)

            return carry1

        lax.fori_loop(0, NSUB, sub_body, 0)

    @pl.when(trips > 0)
    def _():
        pltpu.async_copy(chunk_slice(0), eb0, sem0)

    for k in range(MAXT):
        buf, sem = bufs[k % 2], sems[k % 2]
        nbuf, nsem = bufs[(k + 1) % 2], sems[(k + 1) % 2]

        @pl.when(k + 1 < trips)
        def _():
            pltpu.async_copy(chunk_slice(k + 1), nbuf, nsem)

        @pl.when(k < trips)
        def _():
            pltpu.make_async_copy(chunk_slice(k), buf, sem).wait()
            scan_buf(buf)

    pltpu.sync_copy(accmat, out_hbm.at[pl.ds(wid * 16, 16)])


BR = 10000
NB = N // BR


def _dense_body(x_ref, p_ref, ht_ref, w_ref, xg_ref, wgb_ref, wmb_ref,
                out_ref, facc):
    i = pl.program_id(0)

    @pl.when(i == 0)
    def _():
        facc[...] = jnp.zeros_like(facc)

    s6 = jnp.sum(p_ref[...], axis=0, keepdims=True)[:, :T]

    xaug = jnp.concatenate(
        [x_ref[...], jnp.ones((BR, 1), jnp.float32)], axis=1)
    tot = jnp.zeros((M, 1), jnp.float32)
    for L in range(R + 1):
        hs = lax.dot_general(ht_ref[L], s6, (((1,), (1,)), ((), ())),
                             preferred_element_type=jnp.float32)
        htaug = jnp.concatenate([ht_ref[L], hs], axis=1)
        zt = lax.dot_general(htaug, xaug, (((1,), (1,)), ((), ())),
                             preferred_element_type=jnp.float32)
        e = jnp.exp(jax.nn.sigmoid(zt) * w_ref[0, L])
        den = jnp.sum(e, axis=0, keepdims=True)
        fl = e / den
        tot = tot + jnp.sum(fl, axis=1, keepdims=True)
    facc[0:M, 0:1] = facc[0:M, 0:1] + tot

    @pl.when(i == NB - 1)
    def _():
        f = facc[0:M, 0:1]
        xgaug = jnp.concatenate(
            [xg_ref[...], jnp.ones((1, 1), jnp.float32)], axis=1)
        g = jax.nn.sigmoid(
            lax.dot_general(wgb_ref[...], xgaug, (((1,), (1,)), ((), ())),
                            preferred_element_type=jnp.float32))
        merged = jnp.concatenate(
            [f, g, jnp.ones((1, 1), jnp.float32)], axis=0)
        z3 = lax.dot_general(wmb_ref[...], merged, (((1,), (0,)), ((), ())),
                             preferred_element_type=jnp.float32)
        o3 = jax.nn.softmax(z3, axis=0)
        out_ref[...] = jnp.zeros((8, 128), jnp.float32)
        out_ref[0:3, 0:1] = o3


def kernel(x_member, edge_index, x_group, H, W, Wg, bg, Wm, bm):
    mesh = plsc.VectorSubcoreMesh(core_axis_name="c", subcore_axis_name="s")
    sc_fn = pl.kernel(
        _sc_body,
        mesh=mesh,
        out_type=jax.ShapeDtypeStruct((NW * 16,), jnp.float32),
        scratch_types=[
            pltpu.VMEM((2, CHT), jnp.int32),
            pltpu.VMEM((2, CHT), jnp.int32),
            pltpu.VMEM((8, T), jnp.float32),
            pltpu.VMEM((16,), jnp.float32),
            pltpu.VMEM((32,), jnp.int32),
            pltpu.SemaphoreType.DMA,
            pltpu.SemaphoreType.DMA,
            pltpu.SemaphoreType.DMA,
        ],
        compiler_params=pltpu.CompilerParams(needs_layout_passes=False),
    )
    partials = sc_fn(edge_index, x_member)
    pmat = partials.reshape(NW, 16)

    Ht = H.transpose(0, 2, 1)
    Wgb = jnp.concatenate([Wg, bg[:, None]], axis=1)
    Wmb = jnp.concatenate([Wm, bm[:, None]], axis=1)

    out = pl.pallas_call(
        _dense_body,
        grid=(NB,),
        in_specs=[
            pl.BlockSpec((BR, T), lambda i: (i, 0)),
            pl.BlockSpec((NW, 16), lambda i: (0, 0)),
            pl.BlockSpec((R + 1, M, T), lambda i: (0, 0, 0)),
            pl.BlockSpec((1, R + 1), lambda i: (0, 0)),
            pl.BlockSpec((1, 14), lambda i: (0, 0)),
            pl.BlockSpec((G, 15), lambda i: (0, 0)),
            pl.BlockSpec((3, M + G + 1), lambda i: (0, 0)),
        ],
        out_specs=pl.BlockSpec((8, 128), lambda i: (0, 0)),
        out_shape=jax.ShapeDtypeStruct((8, 128), jnp.float32),
        scratch_shapes=[pltpu.VMEM((16, 128), jnp.float32)],
    )(x_member, pmat, Ht, W.reshape(1, R + 1), x_group, Wgb, Wmb)

    return out[0:3, 0:1].reshape(1, 3)
